# baseline (speedup 1.0000x reference)
import jax
import jax.numpy as jnp
from jax import lax
from jax.experimental import pallas as pl
from jax.experimental.pallas import tpu as pltpu
from jax.experimental.pallas import tpu_sc as plsc

VOCAB = 1000000
D_EMB = 64
B = 4096
L = 200

NW = 32
BLK = B // NW
NQ = 8
NT = 4
KS = 6
KG = 4


def _embed_body(x_hbm, mask_hbm, table_hbm, out_hbm, qx, qm, rows, tbuf,
                qsem, gsem, wsem):
    wid = lax.axis_index("s") * 2 + lax.axis_index("c")
    lane = lax.iota(jnp.int32, 16)
    tdv = []
    ddv = []
    for k in range(4):
        d = k * 16 + lane
        tdv.append(d // 8)
        ddv.append(d % 8)

    def stage(l, b):
        src = l * B + wid * BLK
        pltpu.async_copy(x_hbm.at[pl.ds(src, BLK)], qx[b], qsem[b])
        pltpu.async_copy(mask_hbm.at[pl.ds(src, BLK)], qm[b], qsem[b])

    def stage_wait(l, b):
        src = l * B + wid * BLK
        pltpu.make_async_copy(x_hbm.at[pl.ds(src, BLK)], qx[b],
                              qsem[b]).wait()
        pltpu.make_async_copy(mask_hbm.at[pl.ds(src, BLK)], qm[b],
                              qsem[b]).wait()

    def gather(b):
        pltpu.async_copy(table_hbm.at[qx[b]], rows[b], gsem[b])

    def gather_wait(b):
        pltpu.make_async_copy(table_hbm.at[qx[b]], rows[b], gsem[b]).wait()

    def wout(l, t):
        pltpu.async_copy(tbuf[t].at[:, :, pl.ds(0, BLK)],
                         out_hbm.at[l, :, wid], wsem[t])

    def wout_wait(l, t):
        pltpu.make_async_copy(tbuf[t].at[:, :, pl.ds(0, BLK)],
                              out_hbm.at[l, :, wid], wsem[t]).wait()

    def transpose_mask(b, t):
        tb = tbuf[t]
        rb = rows[b]
        mb = qm[b]

        @pl.loop(0, BLK // 16, unroll=2)
        def _grp(g):
            mvec = mb[pl.ds(g * 16, 16)].astype(jnp.float32)
            r0 = g * 16
            for j in range(16):
                m = mvec[j]
                r = r0 + j
                rvec = jnp.zeros((16,), jnp.int32) + r
                for k in range(4):
                    v = rb[r, pl.ds(k * 16, 16)] * m
                    plsc.store_scatter(tb, [tdv[k], ddv[k], rvec], v)

    for l in range(KS):
        stage(l, l)
    for l in range(KG):
        stage_wait(l, l)
        gather(l)

    @pl.loop(0, L, step=NQ)
    def _ring(l0):
        for i in range(NQ):
            l = l0 + i
            sl = l + KS
            gl = l + KG

            @pl.when(sl < L)
            def _stage():
                stage(sl, (i + KS) % NQ)

            @pl.when(gl < L)
            def _gather():
                stage_wait(gl, (i + KG) % NQ)
                gather((i + KG) % NQ)

            gather_wait(i)

            t = i % NT

            @pl.when(l >= NT)
            def _wdrain():
                wout_wait(l - NT, t)

            transpose_mask(i, t)
            wout(l, t)

    for u in range(NT):
        l = L - NT + u
        wout_wait(l, l % NT)


@jax.jit
def _embed(x_flat, mask_flat, embed_weight):
    mesh = plsc.VectorSubcoreMesh(core_axis_name="c", subcore_axis_name="s")

    def body(x_hbm, mask_hbm, table_hbm, out_hbm, *rest):
        qx = list(rest[:NQ])
        qm = list(rest[NQ:2 * NQ])
        rows = list(rest[2 * NQ:3 * NQ])
        tbuf = list(rest[3 * NQ:3 * NQ + NT])
        sems = rest[3 * NQ + NT:]
        qsem = list(sems[:NQ])
        gsem = list(sems[NQ:2 * NQ])
        wsem = list(sems[2 * NQ:])
        _embed_body(x_hbm, mask_hbm, table_hbm, out_hbm, qx, qm, rows, tbuf,
                    qsem, gsem, wsem)

    f = pl.kernel(
        body,
        out_type=jax.ShapeDtypeStruct((L, 8, NW, 8, BLK), jnp.float32),
        mesh=mesh,
        scratch_types=[pltpu.VMEM((BLK,), jnp.int32)] * NQ
          + [pltpu.VMEM((BLK,), jnp.int32)] * NQ
          + [pltpu.VMEM((BLK, D_EMB), jnp.float32)] * NQ
          + [pltpu.VMEM((8, 8, 133), jnp.float32)] * NT
          + [pltpu.SemaphoreType.DMA] * (2 * NQ + NT),
        compiler_params=pltpu.CompilerParams(
            needs_layout_passes=False, use_tc_tiling_on_sc=False),
    )
    return f(x_flat, mask_flat, embed_weight)


def kernel(x, mask, embed_weight):
    xt = x.T.reshape(-1).astype(jnp.int32)
    mt = mask.T.reshape(-1).astype(jnp.int32)
    out5 = _embed(xt, mt, embed_weight)
    return out5.transpose(2, 4, 0, 1, 3).reshape(B, L, D_EMB)


# speedup vs baseline: 1.0185x; 1.0185x over previous
import jax
import jax.numpy as jnp
from jax import lax
from jax.experimental import pallas as pl
from jax.experimental.pallas import tpu as pltpu
from jax.experimental.pallas import tpu_sc as plsc

VOCAB = 1000000
D_EMB = 64
B = 4096
L = 200

NW = 32
BLK = B // NW
NQ = 8
NT = 4
KS = 6
KG = 4


def _embed_body(x_hbm, mask_hbm, table_hbm, out_hbm, qx, qm, rows, tbuf,
                qsem, gsem, wsem):
    wid = lax.axis_index("s") * 2 + lax.axis_index("c")
    lane = lax.iota(jnp.int32, 16)
    tdv = []
    ddv = []
    for k in range(4):
        d = k * 16 + lane
        tdv.append(d // 8)
        ddv.append(d % 8)

    def stage(l, b):
        src = l * B + wid * BLK
        pltpu.async_copy(x_hbm.at[pl.ds(src, BLK)], qx[b], qsem[b])
        pltpu.async_copy(mask_hbm.at[pl.ds(src, BLK)], qm[b], qsem[b])

    def stage_wait(l, b):
        src = l * B + wid * BLK
        pltpu.make_async_copy(x_hbm.at[pl.ds(src, BLK)], qx[b],
                              qsem[b]).wait()
        pltpu.make_async_copy(mask_hbm.at[pl.ds(src, BLK)], qm[b],
                              qsem[b]).wait()

    def gather(b):
        pltpu.async_copy(table_hbm.at[qx[b]], rows[b], gsem[b])

    def gather_wait(b):
        pltpu.make_async_copy(table_hbm.at[qx[b]], rows[b], gsem[b]).wait()

    def wout(l, t):
        pltpu.async_copy(tbuf[t].at[:, :, pl.ds(0, BLK)],
                         out_hbm.at[l, :, wid], wsem[t])

    def wout_wait(l, t):
        pltpu.make_async_copy(tbuf[t].at[:, :, pl.ds(0, BLK)],
                              out_hbm.at[l, :, wid], wsem[t]).wait()

    def transpose_mask(b, t):
        tb = tbuf[t]
        rb = rows[b]
        mb = qm[b]

        @pl.loop(0, BLK // 16)
        def _grp(g):
            mvec = mb[pl.ds(g * 16, 16)].astype(jnp.float32)
            r0 = g * 16
            for j in range(16):
                m = mvec[j]
                r = r0 + j
                rvec = jnp.zeros((16,), jnp.int32) + r
                for k in range(4):
                    v = rb[r, pl.ds(k * 16, 16)] * m
                    plsc.store_scatter(tb, [tdv[k], ddv[k], rvec], v)

    for l in range(KS):
        stage(l, l)
    for l in range(KG):
        stage_wait(l, l)
        gather(l)

    @pl.loop(0, L, step=NQ)
    def _ring(l0):
        for i in range(NQ):
            l = l0 + i
            sl = l + KS
            gl = l + KG

            @pl.when(sl < L)
            def _stage():
                stage(sl, (i + KS) % NQ)

            @pl.when(gl < L)
            def _gather():
                stage_wait(gl, (i + KG) % NQ)
                gather((i + KG) % NQ)

            gather_wait(i)

            t = i % NT

            @pl.when(l >= NT)
            def _wdrain():
                wout_wait(l - NT, t)

            transpose_mask(i, t)
            wout(l, t)

    for u in range(NT):
        l = L - NT + u
        wout_wait(l, l % NT)


@jax.jit
def _embed(x_flat, mask_flat, embed_weight):
    mesh = plsc.VectorSubcoreMesh(core_axis_name="c", subcore_axis_name="s")

    def body(x_hbm, mask_hbm, table_hbm, out_hbm, *rest):
        qx = list(rest[:NQ])
        qm = list(rest[NQ:2 * NQ])
        rows = list(rest[2 * NQ:3 * NQ])
        tbuf = list(rest[3 * NQ:3 * NQ + NT])
        sems = rest[3 * NQ + NT:]
        qsem = list(sems[:NQ])
        gsem = list(sems[NQ:2 * NQ])
        wsem = list(sems[2 * NQ:])
        _embed_body(x_hbm, mask_hbm, table_hbm, out_hbm, qx, qm, rows, tbuf,
                    qsem, gsem, wsem)

    f = pl.kernel(
        body,
        out_type=jax.ShapeDtypeStruct((L, 8, NW, 8, BLK), jnp.float32),
        mesh=mesh,
        scratch_types=[pltpu.VMEM((BLK,), jnp.int32)] * NQ
          + [pltpu.VMEM((BLK,), jnp.int32)] * NQ
          + [pltpu.VMEM((BLK, D_EMB), jnp.float32)] * NQ
          + [pltpu.VMEM((8, 8, 133), jnp.float32)] * NT
          + [pltpu.SemaphoreType.DMA] * (2 * NQ + NT),
        compiler_params=pltpu.CompilerParams(
            needs_layout_passes=False, use_tc_tiling_on_sc=False),
    )
    return f(x_flat, mask_flat, embed_weight)


def kernel(x, mask, embed_weight):
    xt = x.T.reshape(-1).astype(jnp.int32)
    mt = mask.T.reshape(-1).astype(jnp.int32)
    out5 = _embed(xt, mt, embed_weight)
    return out5.transpose(2, 4, 0, 1, 3).reshape(B, L, D_EMB)
